# Initial kernel scaffold; baseline (speedup 1.0000x reference)
import functools

import jax
import jax.numpy as jnp
from jax import lax
from jax.experimental import pallas as pl
from jax.experimental.pallas import tpu as pltpu
from jax.experimental.pallas import tpu_sc as plsc

B, L, V, D = 16384, 200, 1000000, 32
NC, NS = 2, 16
NW = NC * NS
ROWS_PER_W = B // NW
LANES = 16
SPLIT0, SPLIT1 = 104, 96


def _maxpool_sc(x, emb):
  mesh = plsc.VectorSubcoreMesh(core_axis_name="c", subcore_axis_name="s")

  @functools.partial(
      pl.kernel,
      out_type=jax.ShapeDtypeStruct((B, D), jnp.float32),
      mesh=mesh,
      scratch_types=[
          pltpu.VMEM((L,), jnp.int32),
          pltpu.VMEM((L, D), jnp.float32),
          pltpu.VMEM((ROWS_PER_W, D), jnp.float32),
          pltpu.SemaphoreType.DMA,
      ],
  )
  def k(x_hbm, emb_hbm, out_hbm, idx_v, rows_v, out_v, sem):
    wid = lax.axis_index("s") * NC + lax.axis_index("c")
    base = wid * ROWS_PER_W

    def body(b, carry):
      pltpu.sync_copy(x_hbm.at[base + b], idx_v)
      cp0 = pltpu.async_copy(
          emb_hbm.at[idx_v.at[pl.ds(0, SPLIT0)]],
          rows_v.at[pl.ds(0, SPLIT0)], sem)
      cp1 = pltpu.async_copy(
          emb_hbm.at[idx_v.at[pl.ds(SPLIT0, SPLIT1)]],
          rows_v.at[pl.ds(SPLIT0, SPLIT1)], sem)
      cp0.wait()
      cp1.wait()

      def jbody(j, accs):
        a0, a1 = accs
        return (jnp.maximum(a0, rows_v[j, pl.ds(0, LANES)]),
                jnp.maximum(a1, rows_v[j, pl.ds(LANES, LANES)]))

      acc0, acc1 = lax.fori_loop(
          1, L, jbody,
          (rows_v[0, pl.ds(0, LANES)], rows_v[0, pl.ds(LANES, LANES)]))
      out_v[b, pl.ds(0, LANES)] = acc0
      out_v[b, pl.ds(LANES, LANES)] = acc1
      return carry

    lax.fori_loop(0, ROWS_PER_W, body, 0)
    pltpu.sync_copy(out_v, out_hbm.at[pl.ds(base, ROWS_PER_W)])

  return k(x, emb)


def _head_tc(set_vec, W_h, b_h, W_cls, b_cls, W_reg, b_reg):
  blk = 2048
  grid = (B // blk,)

  def body(s_ref, wh_ref, bh_ref, wc_ref, bc_ref, wr_ref, br_ref,
           cls_ref, reg_ref):
    s = s_ref[...]
    h = jnp.dot(s, wh_ref[...], preferred_element_type=jnp.float32)
    h = h + bh_ref[...]
    h = jnp.where(h > 0, h, 0.01 * h)
    c = jnp.dot(h, wc_ref[...], preferred_element_type=jnp.float32)
    c = jax.nn.sigmoid(c + bc_ref[...])
    r = jnp.dot(h, wr_ref[...], preferred_element_type=jnp.float32)
    cls_ref[...] = c
    reg_ref[...] = r + br_ref[...]

  cls, reg = pl.pallas_call(
      body,
      grid=grid,
      in_specs=[
          pl.BlockSpec((blk, D), lambda i: (i, 0)),
          pl.BlockSpec((D, D), lambda i: (0, 0)),
          pl.BlockSpec((1, D), lambda i: (0, 0)),
          pl.BlockSpec((D, 1), lambda i: (0, 0)),
          pl.BlockSpec((1, 1), lambda i: (0, 0)),
          pl.BlockSpec((D, 1), lambda i: (0, 0)),
          pl.BlockSpec((1, 1), lambda i: (0, 0)),
      ],
      out_specs=[
          pl.BlockSpec((blk, 1), lambda i: (i, 0)),
          pl.BlockSpec((blk, 1), lambda i: (i, 0)),
      ],
      out_shape=[
          jax.ShapeDtypeStruct((B, 1), jnp.float32),
          jax.ShapeDtypeStruct((B, 1), jnp.float32),
      ],
  )(set_vec, W_h, b_h.reshape(1, D), W_cls, b_cls.reshape(1, 1),
    W_reg, b_reg.reshape(1, 1))
  return cls, reg


def kernel(x, emb, W_h, b_h, W_cls, b_cls, W_reg, b_reg):
  set_vec = _maxpool_sc(x.astype(jnp.int32), emb)
  return _head_tc(set_vec, W_h, b_h, W_cls, b_cls, W_reg, b_reg)


# speedup vs baseline: 7.2592x; 7.2592x over previous
import functools

import jax
import jax.numpy as jnp
from jax import lax
from jax.experimental import pallas as pl
from jax.experimental.pallas import tpu as pltpu
from jax.experimental.pallas import tpu_sc as plsc

B, L, V, D = 16384, 200, 1000000, 32
NC, NS = 2, 16
NW = NC * NS
ROWS_PER_W = B // NW
LANES = 16
SPLIT0, SPLIT1 = 104, 96


def _maxpool_sc(x, emb):
  mesh = plsc.VectorSubcoreMesh(core_axis_name="c", subcore_axis_name="s")

  @functools.partial(
      pl.kernel,
      out_type=jax.ShapeDtypeStruct((B, D), jnp.float32),
      mesh=mesh,
      scratch_types=[
          pltpu.VMEM((L,), jnp.int32),
          pltpu.VMEM((L, D), jnp.float32),
          pltpu.VMEM((ROWS_PER_W, D), jnp.float32),
          pltpu.SemaphoreType.DMA,
      ],
      compiler_params=pltpu.CompilerParams(use_tc_tiling_on_sc=False),
  )
  def k(x_hbm, emb_hbm, out_hbm, idx_v, rows_v, out_v, sem):
    wid = lax.axis_index("s") * NC + lax.axis_index("c")
    base = wid * ROWS_PER_W

    def body(b, carry):
      pltpu.sync_copy(x_hbm.at[base + b], idx_v)
      cp0 = pltpu.async_copy(
          emb_hbm.at[idx_v.at[pl.ds(0, SPLIT0)]],
          rows_v.at[pl.ds(0, SPLIT0)], sem)
      cp1 = pltpu.async_copy(
          emb_hbm.at[idx_v.at[pl.ds(SPLIT0, SPLIT1)]],
          rows_v.at[pl.ds(SPLIT0, SPLIT1)], sem)
      cp0.wait()
      cp1.wait()

      def jbody(j, accs):
        a0, a1 = accs
        return (jnp.maximum(a0, rows_v[j, pl.ds(0, LANES)]),
                jnp.maximum(a1, rows_v[j, pl.ds(LANES, LANES)]))

      acc0, acc1 = lax.fori_loop(
          1, L, jbody,
          (rows_v[0, pl.ds(0, LANES)], rows_v[0, pl.ds(LANES, LANES)]))
      out_v[b, pl.ds(0, LANES)] = acc0
      out_v[b, pl.ds(LANES, LANES)] = acc1
      return carry

    lax.fori_loop(0, ROWS_PER_W, body, 0)
    pltpu.sync_copy(out_v, out_hbm.at[pl.ds(base, ROWS_PER_W)])

  return k(x, emb)


def _head_tc(set_vec, W_h, b_h, W_cls, b_cls, W_reg, b_reg):
  blk = 2048
  grid = (B // blk,)

  def body(s_ref, wh_ref, bh_ref, wc_ref, bc_ref, wr_ref, br_ref,
           cls_ref, reg_ref):
    s = s_ref[...]
    h = jnp.dot(s, wh_ref[...], preferred_element_type=jnp.float32)
    h = h + bh_ref[...]
    h = jnp.where(h > 0, h, 0.01 * h)
    c = jnp.dot(h, wc_ref[...], preferred_element_type=jnp.float32)
    c = jax.nn.sigmoid(c + bc_ref[...])
    r = jnp.dot(h, wr_ref[...], preferred_element_type=jnp.float32)
    cls_ref[...] = c
    reg_ref[...] = r + br_ref[...]

  cls, reg = pl.pallas_call(
      body,
      grid=grid,
      in_specs=[
          pl.BlockSpec((blk, D), lambda i: (i, 0)),
          pl.BlockSpec((D, D), lambda i: (0, 0)),
          pl.BlockSpec((1, D), lambda i: (0, 0)),
          pl.BlockSpec((D, 1), lambda i: (0, 0)),
          pl.BlockSpec((1, 1), lambda i: (0, 0)),
          pl.BlockSpec((D, 1), lambda i: (0, 0)),
          pl.BlockSpec((1, 1), lambda i: (0, 0)),
      ],
      out_specs=[
          pl.BlockSpec((blk, 1), lambda i: (i, 0)),
          pl.BlockSpec((blk, 1), lambda i: (i, 0)),
      ],
      out_shape=[
          jax.ShapeDtypeStruct((B, 1), jnp.float32),
          jax.ShapeDtypeStruct((B, 1), jnp.float32),
      ],
  )(set_vec, W_h, b_h.reshape(1, D), W_cls, b_cls.reshape(1, 1),
    W_reg, b_reg.reshape(1, 1))
  return cls, reg


def kernel(x, emb, W_h, b_h, W_cls, b_cls, W_reg, b_reg):
  set_vec = _maxpool_sc(x.astype(jnp.int32), emb)
  return _head_tc(set_vec, W_h, b_h, W_cls, b_cls, W_reg, b_reg)


# speedup vs baseline: 14.9617x; 2.0611x over previous
import functools

import jax
import jax.numpy as jnp
from jax import lax
from jax.experimental import pallas as pl
from jax.experimental.pallas import tpu as pltpu
from jax.experimental.pallas import tpu_sc as plsc

B, L, V, D = 16384, 200, 1000000, 32
NC, NS = 2, 16
NW = NC * NS
ROWS_PER_W = B // NW
LANES = 16
SPLIT0, SPLIT1 = 104, 96


G = 4
NG = ROWS_PER_W // G
UNROLL = 8


def _maxpool_sc(x, emb):
  mesh = plsc.VectorSubcoreMesh(core_axis_name="c", subcore_axis_name="s")

  @functools.partial(
      pl.kernel,
      out_type=jax.ShapeDtypeStruct((B, D), jnp.float32),
      mesh=mesh,
      scratch_types=[
          pltpu.VMEM((2, G, L), jnp.int32),
          pltpu.VMEM((2, G, L, D), jnp.float32),
          pltpu.VMEM((ROWS_PER_W, D), jnp.float32),
          pltpu.SemaphoreType.DMA,
          pltpu.SemaphoreType.DMA,
      ],
      compiler_params=pltpu.CompilerParams(use_tc_tiling_on_sc=False),
  )
  def k(x_hbm, emb_hbm, out_hbm, idx_v, rows_v, out_v, sem0, sem1):
    wid = lax.axis_index("s") * NC + lax.axis_index("c")
    base = wid * ROWS_PER_W
    sems = (sem0, sem1)

    def fire(g, p):
      sem = sems[p]
      pltpu.sync_copy(x_hbm.at[pl.ds(base + g * G, G)], idx_v.at[p])
      for r in range(G):
        pltpu.async_copy(
            emb_hbm.at[idx_v.at[p, r, pl.ds(0, SPLIT0)]],
            rows_v.at[p, r, pl.ds(0, SPLIT0)], sem)
        pltpu.async_copy(
            emb_hbm.at[idx_v.at[p, r, pl.ds(SPLIT0, SPLIT1)]],
            rows_v.at[p, r, pl.ds(SPLIT0, SPLIT1)], sem)

    def drain(p):
      sem = sems[p]
      for r in range(G):
        pltpu.make_async_copy(
            emb_hbm.at[idx_v.at[p, r, pl.ds(0, SPLIT0)]],
            rows_v.at[p, r, pl.ds(0, SPLIT0)], sem).wait()
        pltpu.make_async_copy(
            emb_hbm.at[idx_v.at[p, r, pl.ds(SPLIT0, SPLIT1)]],
            rows_v.at[p, r, pl.ds(SPLIT0, SPLIT1)], sem).wait()

    neg_inf = jnp.full((LANES,), -jnp.inf, jnp.float32)

    def compute(g, p):
      for r in range(G):
        def jbody(t, accs, r=r):
          a0, a1 = accs
          j = t * UNROLL
          for u in range(UNROLL):
            a0 = jnp.maximum(a0, rows_v[p, r, j + u, pl.ds(0, LANES)])
            a1 = jnp.maximum(a1, rows_v[p, r, j + u, pl.ds(LANES, LANES)])
          return (a0, a1)

        acc0, acc1 = lax.fori_loop(0, L // UNROLL, jbody, (neg_inf, neg_inf))
        out_v[g * G + r, pl.ds(0, LANES)] = acc0
        out_v[g * G + r, pl.ds(LANES, LANES)] = acc1

    fire(0, 0)

    def pair_body(gg, carry):
      g0 = 2 * gg
      fire(g0 + 1, 1)
      drain(0)
      compute(g0, 0)

      @pl.when(g0 + 2 < NG)
      def _():
        fire(g0 + 2, 0)

      drain(1)
      compute(g0 + 1, 1)
      return carry

    lax.fori_loop(0, NG // 2, pair_body, 0)
    pltpu.sync_copy(out_v, out_hbm.at[pl.ds(base, ROWS_PER_W)])

  return k(x, emb)


def _head_tc(set_vec, W_h, b_h, W_cls, b_cls, W_reg, b_reg):
  blk = 2048
  grid = (B // blk,)

  def body(s_ref, wh_ref, bh_ref, wc_ref, bc_ref, wr_ref, br_ref,
           cls_ref, reg_ref):
    s = s_ref[...]
    h = jnp.dot(s, wh_ref[...], preferred_element_type=jnp.float32)
    h = h + bh_ref[...]
    h = jnp.where(h > 0, h, 0.01 * h)
    c = jnp.dot(h, wc_ref[...], preferred_element_type=jnp.float32)
    c = jax.nn.sigmoid(c + bc_ref[...])
    r = jnp.dot(h, wr_ref[...], preferred_element_type=jnp.float32)
    cls_ref[...] = c
    reg_ref[...] = r + br_ref[...]

  cls, reg = pl.pallas_call(
      body,
      grid=grid,
      in_specs=[
          pl.BlockSpec((blk, D), lambda i: (i, 0)),
          pl.BlockSpec((D, D), lambda i: (0, 0)),
          pl.BlockSpec((1, D), lambda i: (0, 0)),
          pl.BlockSpec((D, 1), lambda i: (0, 0)),
          pl.BlockSpec((1, 1), lambda i: (0, 0)),
          pl.BlockSpec((D, 1), lambda i: (0, 0)),
          pl.BlockSpec((1, 1), lambda i: (0, 0)),
      ],
      out_specs=[
          pl.BlockSpec((blk, 1), lambda i: (i, 0)),
          pl.BlockSpec((blk, 1), lambda i: (i, 0)),
      ],
      out_shape=[
          jax.ShapeDtypeStruct((B, 1), jnp.float32),
          jax.ShapeDtypeStruct((B, 1), jnp.float32),
      ],
  )(set_vec, W_h, b_h.reshape(1, D), W_cls, b_cls.reshape(1, 1),
    W_reg, b_reg.reshape(1, 1))
  return cls, reg


def kernel(x, emb, W_h, b_h, W_cls, b_cls, W_reg, b_reg):
  set_vec = _maxpool_sc(x.astype(jnp.int32), emb)
  return _head_tc(set_vec, W_h, b_h, W_cls, b_cls, W_reg, b_reg)


# speedup vs baseline: 21.2856x; 1.4227x over previous
import functools

import jax
import jax.numpy as jnp
from jax import lax
from jax.experimental import pallas as pl
from jax.experimental.pallas import tpu as pltpu
from jax.experimental.pallas import tpu_sc as plsc

B, L, V, D = 16384, 200, 1000000, 32
NC, NS = 2, 16
NW = NC * NS
ROWS_PER_W = B // NW
LANES = 16
SPLIT0, SPLIT1 = 104, 96
G = 4
NG = ROWS_PER_W // G
GL = G * L
NVEC = GL // LANES
UNROLL = 8

BC = 8192
CH = BC // 4
NBLK = (V + BC - 1) // BC
VPAD = NBLK * BC


def _format_table(emb):
  embT = emb.T

  def body(in_ref, out_ref):
    t = in_ref[...]
    z = jnp.concatenate(
        [t[:, CH * k:CH * (k + 1)].T for k in range(4)], axis=1)
    out_ref[...] = z[None]

  out = pl.pallas_call(
      body,
      grid=(NBLK,),
      in_specs=[pl.BlockSpec((D, BC), lambda i: (0, i))],
      out_specs=pl.BlockSpec((1, CH, 128), lambda i: (i, 0, 0)),
      out_shape=jax.ShapeDtypeStruct((NBLK, CH, 128), jnp.float32),
  )(embT)
  return out.reshape(VPAD, D)


def _remap(i):
  return (i & -8192) | ((i & 2047) << 2) | ((i >> 11) & 3)


def _maxpool_sc(xf, table):
  mesh = plsc.VectorSubcoreMesh(core_axis_name="c", subcore_axis_name="s")

  @functools.partial(
      pl.kernel,
      out_type=jax.ShapeDtypeStruct((B, D), jnp.float32),
      mesh=mesh,
      scratch_types=[
          pltpu.VMEM((2, GL), jnp.int32),
          pltpu.VMEM((2, G, L, D), jnp.float32),
          pltpu.VMEM((ROWS_PER_W, D), jnp.float32),
          pltpu.SemaphoreType.DMA,
          pltpu.SemaphoreType.DMA,
      ],
      compiler_params=pltpu.CompilerParams(use_tc_tiling_on_sc=False),
  )
  def k(xf_hbm, tab_hbm, out_hbm, idx_v, rows_v, out_v, sem0, sem1):
    wid = lax.axis_index("s") * NC + lax.axis_index("c")
    base = wid * ROWS_PER_W
    fbase = base * L
    sems = (sem0, sem1)

    def fire(g, p):
      sem = sems[p]
      pltpu.sync_copy(xf_hbm.at[pl.ds(fbase + g * GL, GL)], idx_v.at[p])

      def tbody(t, carry):
        for u in range(5):
          o = (t * 5 + u) * LANES
          idx_v[p, pl.ds(o, LANES)] = _remap(idx_v[p, pl.ds(o, LANES)])
        return carry

      lax.fori_loop(0, NVEC // 5, tbody, 0)
      for r in range(G):
        pltpu.async_copy(
            tab_hbm.at[idx_v.at[p, pl.ds(r * L, SPLIT0)]],
            rows_v.at[p, r, pl.ds(0, SPLIT0)], sem)
        pltpu.async_copy(
            tab_hbm.at[idx_v.at[p, pl.ds(r * L + SPLIT0, SPLIT1)]],
            rows_v.at[p, r, pl.ds(SPLIT0, SPLIT1)], sem)

    def drain(p):
      sem = sems[p]
      for r in range(G):
        pltpu.make_async_copy(
            tab_hbm.at[idx_v.at[p, pl.ds(r * L, SPLIT0)]],
            rows_v.at[p, r, pl.ds(0, SPLIT0)], sem).wait()
        pltpu.make_async_copy(
            tab_hbm.at[idx_v.at[p, pl.ds(r * L + SPLIT0, SPLIT1)]],
            rows_v.at[p, r, pl.ds(SPLIT0, SPLIT1)], sem).wait()

    neg_inf = jnp.full((LANES,), -jnp.inf, jnp.float32)

    def compute(g, p):
      for r in range(G):
        def jbody(t, accs, r=r):
          a0, a1 = accs
          j = t * UNROLL
          for u in range(UNROLL):
            a0 = jnp.maximum(a0, rows_v[p, r, j + u, pl.ds(0, LANES)])
            a1 = jnp.maximum(a1, rows_v[p, r, j + u, pl.ds(LANES, LANES)])
          return (a0, a1)

        acc0, acc1 = lax.fori_loop(0, L // UNROLL, jbody, (neg_inf, neg_inf))
        out_v[g * G + r, pl.ds(0, LANES)] = acc0
        out_v[g * G + r, pl.ds(LANES, LANES)] = acc1

    fire(0, 0)

    def pair_body(gg, carry):
      g0 = 2 * gg
      fire(g0 + 1, 1)
      drain(0)
      compute(g0, 0)

      @pl.when(g0 + 2 < NG)
      def _():
        fire(g0 + 2, 0)

      drain(1)
      compute(g0 + 1, 1)
      return carry

    lax.fori_loop(0, NG // 2, pair_body, 0)
    pltpu.sync_copy(out_v, out_hbm.at[pl.ds(base, ROWS_PER_W)])

  return k(xf, table)


def _head_tc(set_vec, W_h, b_h, W_cls, b_cls, W_reg, b_reg):
  blk = 2048
  grid = (B // blk,)

  def body(s_ref, wh_ref, bh_ref, wc_ref, bc_ref, wr_ref, br_ref,
           cls_ref, reg_ref):
    s = s_ref[...]
    h = jnp.dot(s, wh_ref[...], preferred_element_type=jnp.float32)
    h = h + bh_ref[...]
    h = jnp.where(h > 0, h, 0.01 * h)
    c = jnp.dot(h, wc_ref[...], preferred_element_type=jnp.float32)
    c = jax.nn.sigmoid(c + bc_ref[...])
    r = jnp.dot(h, wr_ref[...], preferred_element_type=jnp.float32)
    cls_ref[...] = c
    reg_ref[...] = r + br_ref[...]

  cls, reg = pl.pallas_call(
      body,
      grid=grid,
      in_specs=[
          pl.BlockSpec((blk, D), lambda i: (i, 0)),
          pl.BlockSpec((D, D), lambda i: (0, 0)),
          pl.BlockSpec((1, D), lambda i: (0, 0)),
          pl.BlockSpec((D, 1), lambda i: (0, 0)),
          pl.BlockSpec((1, 1), lambda i: (0, 0)),
          pl.BlockSpec((D, 1), lambda i: (0, 0)),
          pl.BlockSpec((1, 1), lambda i: (0, 0)),
      ],
      out_specs=[
          pl.BlockSpec((blk, 1), lambda i: (i, 0)),
          pl.BlockSpec((blk, 1), lambda i: (i, 0)),
      ],
      out_shape=[
          jax.ShapeDtypeStruct((B, 1), jnp.float32),
          jax.ShapeDtypeStruct((B, 1), jnp.float32),
      ],
  )(set_vec, W_h, b_h.reshape(1, D), W_cls, b_cls.reshape(1, 1),
    W_reg, b_reg.reshape(1, 1))
  return cls, reg


def kernel(x, emb, W_h, b_h, W_cls, b_cls, W_reg, b_reg):
  table = _format_table(emb)
  xf = x.astype(jnp.int32).reshape(B * L)
  set_vec = _maxpool_sc(xf, table)
  return _head_tc(set_vec, W_h, b_h, W_cls, b_cls, W_reg, b_reg)


# speedup vs baseline: 26.5959x; 1.2495x over previous
import functools

import jax
import jax.numpy as jnp
from jax import lax
from jax.experimental import pallas as pl
from jax.experimental.pallas import tpu as pltpu
from jax.experimental.pallas import tpu_sc as plsc

B, L, V, D = 16384, 200, 1000000, 32
NC, NS = 2, 16
NW = NC * NS
ROWS_PER_W = B // NW
LANES = 16
SPLIT0, SPLIT1 = 104, 96
G = 4
NG = ROWS_PER_W // G
GL = G * L
NVEC = GL // LANES
UNROLL = 8

BC = 8192
CH = BC // 4
NBLK = (V + BC - 1) // BC
VPAD = NBLK * BC


def _format_table(emb):
  embT = emb.T

  def body(in_ref, out_ref):
    t = in_ref[...]
    t4 = jnp.concatenate(
        [t[:, CH * k:CH * (k + 1)] for k in range(4)], axis=0)
    out_ref[...] = t4.T[None]

  out = pl.pallas_call(
      body,
      grid=(NBLK,),
      in_specs=[pl.BlockSpec((D, BC), lambda i: (0, i))],
      out_specs=pl.BlockSpec((1, CH, 128), lambda i: (i, 0, 0)),
      out_shape=jax.ShapeDtypeStruct((NBLK, CH, 128), jnp.float32),
  )(embT)
  return out.reshape(VPAD, D)


def _remap(i):
  return (i & -8192) | ((i & 2047) << 2) | ((i >> 11) & 3)


def _maxpool_sc(xf, table):
  mesh = plsc.VectorSubcoreMesh(core_axis_name="c", subcore_axis_name="s")

  @functools.partial(
      pl.kernel,
      out_type=jax.ShapeDtypeStruct((B, D), jnp.float32),
      mesh=mesh,
      scratch_types=[
          pltpu.VMEM((2, GL), jnp.int32),
          pltpu.VMEM((2, G, L, D), jnp.float32),
          pltpu.VMEM((ROWS_PER_W, D), jnp.float32),
          pltpu.SemaphoreType.DMA,
          pltpu.SemaphoreType.DMA,
      ],
      compiler_params=pltpu.CompilerParams(use_tc_tiling_on_sc=False),
  )
  def k(xf_hbm, tab_hbm, out_hbm, idx_v, rows_v, out_v, sem0, sem1):
    wid = lax.axis_index("s") * NC + lax.axis_index("c")
    base = wid * ROWS_PER_W
    fbase = base * L
    sems = (sem0, sem1)

    def fire(g, p):
      sem = sems[p]
      pltpu.sync_copy(xf_hbm.at[pl.ds(fbase + g * GL, GL)], idx_v.at[p])

      def tbody(t, carry):
        for u in range(5):
          o = (t * 5 + u) * LANES
          idx_v[p, pl.ds(o, LANES)] = _remap(idx_v[p, pl.ds(o, LANES)])
        return carry

      lax.fori_loop(0, NVEC // 5, tbody, 0)
      for r in range(G):
        pltpu.async_copy(
            tab_hbm.at[idx_v.at[p, pl.ds(r * L, SPLIT0)]],
            rows_v.at[p, r, pl.ds(0, SPLIT0)], sem)
        pltpu.async_copy(
            tab_hbm.at[idx_v.at[p, pl.ds(r * L + SPLIT0, SPLIT1)]],
            rows_v.at[p, r, pl.ds(SPLIT0, SPLIT1)], sem)

    def drain(p):
      sem = sems[p]
      for r in range(G):
        pltpu.make_async_copy(
            tab_hbm.at[idx_v.at[p, pl.ds(r * L, SPLIT0)]],
            rows_v.at[p, r, pl.ds(0, SPLIT0)], sem).wait()
        pltpu.make_async_copy(
            tab_hbm.at[idx_v.at[p, pl.ds(r * L + SPLIT0, SPLIT1)]],
            rows_v.at[p, r, pl.ds(SPLIT0, SPLIT1)], sem).wait()

    neg_inf = jnp.full((LANES,), -jnp.inf, jnp.float32)

    def compute(g, p):
      for r in range(G):
        def jbody(t, accs, r=r):
          a0, a1 = accs
          j = t * UNROLL
          for u in range(UNROLL):
            a0 = jnp.maximum(a0, rows_v[p, r, j + u, pl.ds(0, LANES)])
            a1 = jnp.maximum(a1, rows_v[p, r, j + u, pl.ds(LANES, LANES)])
          return (a0, a1)

        acc0, acc1 = lax.fori_loop(0, L // UNROLL, jbody, (neg_inf, neg_inf))
        out_v[g * G + r, pl.ds(0, LANES)] = acc0
        out_v[g * G + r, pl.ds(LANES, LANES)] = acc1

    fire(0, 0)

    def pair_body(gg, carry):
      g0 = 2 * gg
      fire(g0 + 1, 1)
      drain(0)
      compute(g0, 0)

      @pl.when(g0 + 2 < NG)
      def _():
        fire(g0 + 2, 0)

      drain(1)
      compute(g0 + 1, 1)
      return carry

    lax.fori_loop(0, NG // 2, pair_body, 0)
    pltpu.sync_copy(out_v, out_hbm.at[pl.ds(base, ROWS_PER_W)])

  return k(xf, table)


def _head_tc(set_vec, W_h, b_h, W_cls, b_cls, W_reg, b_reg):
  blk = 2048
  grid = (B // blk,)

  def body(s_ref, wh_ref, bh_ref, wc_ref, bc_ref, wr_ref, br_ref,
           cls_ref, reg_ref):
    s = s_ref[...]
    h = jnp.dot(s, wh_ref[...], preferred_element_type=jnp.float32)
    h = h + bh_ref[...]
    h = jnp.where(h > 0, h, 0.01 * h)
    c = jnp.dot(h, wc_ref[...], preferred_element_type=jnp.float32)
    c = jax.nn.sigmoid(c + bc_ref[...])
    r = jnp.dot(h, wr_ref[...], preferred_element_type=jnp.float32)
    cls_ref[...] = c
    reg_ref[...] = r + br_ref[...]

  cls, reg = pl.pallas_call(
      body,
      grid=grid,
      in_specs=[
          pl.BlockSpec((blk, D), lambda i: (i, 0)),
          pl.BlockSpec((D, D), lambda i: (0, 0)),
          pl.BlockSpec((1, D), lambda i: (0, 0)),
          pl.BlockSpec((D, 1), lambda i: (0, 0)),
          pl.BlockSpec((1, 1), lambda i: (0, 0)),
          pl.BlockSpec((D, 1), lambda i: (0, 0)),
          pl.BlockSpec((1, 1), lambda i: (0, 0)),
      ],
      out_specs=[
          pl.BlockSpec((blk, 1), lambda i: (i, 0)),
          pl.BlockSpec((blk, 1), lambda i: (i, 0)),
      ],
      out_shape=[
          jax.ShapeDtypeStruct((B, 1), jnp.float32),
          jax.ShapeDtypeStruct((B, 1), jnp.float32),
      ],
  )(set_vec, W_h, b_h.reshape(1, D), W_cls, b_cls.reshape(1, 1),
    W_reg, b_reg.reshape(1, 1))
  return cls, reg


def kernel(x, emb, W_h, b_h, W_cls, b_cls, W_reg, b_reg):
  table = _format_table(emb)
  xf = x.astype(jnp.int32).reshape(B * L)
  set_vec = _maxpool_sc(xf, table)
  return _head_tc(set_vec, W_h, b_h, W_cls, b_cls, W_reg, b_reg)


# speedup vs baseline: 27.3918x; 1.0299x over previous
import functools

import jax
import jax.numpy as jnp
from jax import lax
from jax.experimental import pallas as pl
from jax.experimental.pallas import tpu as pltpu
from jax.experimental.pallas import tpu_sc as plsc

B, L, V, D = 16384, 200, 1000000, 32
NC, NS = 2, 16
NW = NC * NS
ROWS_PER_W = B // NW
LANES = 16
SPLIT0, SPLIT1 = 104, 96
G = 4
NG = ROWS_PER_W // G
GL = G * L
NVEC = GL // LANES
UNROLL = 25

BC = 8192
CH = BC // 4
NBLK = (V + BC - 1) // BC
VPAD = NBLK * BC


def _format_table(emb):
  embT = emb.T

  def body(in_ref, out_ref):
    t = in_ref[...]
    t4 = jnp.concatenate(
        [t[:, CH * k:CH * (k + 1)] for k in range(4)], axis=0)
    out_ref[...] = t4.T[None]

  out = pl.pallas_call(
      body,
      grid=(NBLK,),
      in_specs=[pl.BlockSpec((D, BC), lambda i: (0, i))],
      out_specs=pl.BlockSpec((1, CH, 128), lambda i: (i, 0, 0)),
      out_shape=jax.ShapeDtypeStruct((NBLK, CH, 128), jnp.float32),
  )(embT)
  return out.reshape(VPAD, D)


def _remap(i):
  return (i & -8192) | ((i & 2047) << 2) | ((i >> 11) & 3)


def _maxpool_sc(xf, table):
  mesh = plsc.VectorSubcoreMesh(core_axis_name="c", subcore_axis_name="s")

  @functools.partial(
      pl.kernel,
      out_type=jax.ShapeDtypeStruct((B, D), jnp.float32),
      mesh=mesh,
      scratch_types=[
          pltpu.VMEM((2, GL), jnp.int32),
          pltpu.VMEM((2, G, L, D), jnp.float32),
          pltpu.VMEM((ROWS_PER_W, D), jnp.float32),
          pltpu.SemaphoreType.DMA,
          pltpu.SemaphoreType.DMA,
      ],
      compiler_params=pltpu.CompilerParams(use_tc_tiling_on_sc=False),
  )
  def k(xf_hbm, tab_hbm, out_hbm, idx_v, rows_v, out_v, sem0, sem1):
    wid = lax.axis_index("s") * NC + lax.axis_index("c")
    base = wid * ROWS_PER_W
    fbase = base * L
    sems = (sem0, sem1)

    def fire(g, p):
      sem = sems[p]
      pltpu.sync_copy(xf_hbm.at[pl.ds(fbase + g * GL, GL)], idx_v.at[p])

      def tbody(t, carry):
        for u in range(5):
          o = (t * 5 + u) * LANES
          idx_v[p, pl.ds(o, LANES)] = _remap(idx_v[p, pl.ds(o, LANES)])
        return carry

      lax.fori_loop(0, NVEC // 5, tbody, 0)
      for r in range(G):
        pltpu.async_copy(
            tab_hbm.at[idx_v.at[p, pl.ds(r * L, SPLIT0)]],
            rows_v.at[p, r, pl.ds(0, SPLIT0)], sem)
        pltpu.async_copy(
            tab_hbm.at[idx_v.at[p, pl.ds(r * L + SPLIT0, SPLIT1)]],
            rows_v.at[p, r, pl.ds(SPLIT0, SPLIT1)], sem)

    def drain(p):
      sem = sems[p]
      for r in range(G):
        pltpu.make_async_copy(
            tab_hbm.at[idx_v.at[p, pl.ds(r * L, SPLIT0)]],
            rows_v.at[p, r, pl.ds(0, SPLIT0)], sem).wait()
        pltpu.make_async_copy(
            tab_hbm.at[idx_v.at[p, pl.ds(r * L + SPLIT0, SPLIT1)]],
            rows_v.at[p, r, pl.ds(SPLIT0, SPLIT1)], sem).wait()

    neg_inf = jnp.full((LANES,), -jnp.inf, jnp.float32)

    def compute(g, p):
      for r in range(G):
        def jbody(t, accs, r=r):
          a0, a1 = accs
          j = t * UNROLL
          for u in range(UNROLL):
            a0 = jnp.maximum(a0, rows_v[p, r, j + u, pl.ds(0, LANES)])
            a1 = jnp.maximum(a1, rows_v[p, r, j + u, pl.ds(LANES, LANES)])
          return (a0, a1)

        acc0, acc1 = lax.fori_loop(0, L // UNROLL, jbody, (neg_inf, neg_inf))
        out_v[g * G + r, pl.ds(0, LANES)] = acc0
        out_v[g * G + r, pl.ds(LANES, LANES)] = acc1

    fire(0, 0)

    def pair_body(gg, carry):
      g0 = 2 * gg
      fire(g0 + 1, 1)
      drain(0)
      compute(g0, 0)

      @pl.when(g0 + 2 < NG)
      def _():
        fire(g0 + 2, 0)

      drain(1)
      compute(g0 + 1, 1)
      return carry

    lax.fori_loop(0, NG // 2, pair_body, 0)
    pltpu.sync_copy(out_v, out_hbm.at[pl.ds(base, ROWS_PER_W)])

  return k(xf, table)


def _head_tc(set4, W_h, b_h, W_cls, b_cls, W_reg, b_reg):
  B4 = B // 4

  def body(s_ref, wh_ref, bh_ref, wc_ref, bc_ref, wr_ref, br_ref,
           cls_ref, reg_ref):
    cs, rs = [], []
    for k in range(4):
      z = s_ref[:, D * k:D * (k + 1)]
      h = jnp.dot(z, wh_ref[...], preferred_element_type=jnp.float32)
      h = h + bh_ref[...]
      h = jnp.where(h > 0, h, 0.01 * h)
      c = jnp.dot(h, wc_ref[...], preferred_element_type=jnp.float32)
      cs.append(jax.nn.sigmoid(c + bc_ref[...]))
      r = jnp.dot(h, wr_ref[...], preferred_element_type=jnp.float32)
      rs.append(r + br_ref[...])
    cls_ref[...] = jnp.concatenate(cs, axis=1)
    reg_ref[...] = jnp.concatenate(rs, axis=1)

  cls, reg = pl.pallas_call(
      body,
      in_specs=[
          pl.BlockSpec((B4, 128), lambda: (0, 0)),
          pl.BlockSpec((D, D), lambda: (0, 0)),
          pl.BlockSpec((1, D), lambda: (0, 0)),
          pl.BlockSpec((D, 1), lambda: (0, 0)),
          pl.BlockSpec((1, 1), lambda: (0, 0)),
          pl.BlockSpec((D, 1), lambda: (0, 0)),
          pl.BlockSpec((1, 1), lambda: (0, 0)),
      ],
      out_specs=[
          pl.BlockSpec((B4, 4), lambda: (0, 0)),
          pl.BlockSpec((B4, 4), lambda: (0, 0)),
      ],
      out_shape=[
          jax.ShapeDtypeStruct((B4, 4), jnp.float32),
          jax.ShapeDtypeStruct((B4, 4), jnp.float32),
      ],
  )(set4, W_h, b_h.reshape(1, D), W_cls, b_cls.reshape(1, 1),
    W_reg, b_reg.reshape(1, 1))
  return cls.reshape(B, 1), reg.reshape(B, 1)


def kernel(x, emb, W_h, b_h, W_cls, b_cls, W_reg, b_reg):
  table = _format_table(emb)
  xf = x.astype(jnp.int32).reshape(B * L)
  set_vec = _maxpool_sc(xf, table)
  set4 = set_vec.reshape(B // 4, 128)
  return _head_tc(set4, W_h, b_h, W_cls, b_cls, W_reg, b_reg)


# speedup vs baseline: 31.5574x; 1.1521x over previous
import functools

import jax
import jax.numpy as jnp
from jax import lax
from jax.experimental import pallas as pl
from jax.experimental.pallas import tpu as pltpu
from jax.experimental.pallas import tpu_sc as plsc

B, L, V, D = 16384, 200, 1000000, 32
DP = D // 2
NC, NS = 2, 16
NW = NC * NS
ROWS_PER_W = B // NW
SPLIT0, SPLIT1 = 104, 96
G = 4
NG = ROWS_PER_W // G
GL = G * L
LANES = 16
NVEC = GL // LANES
UNROLL = 25

BC = 8192
CH = BC // 8
NBLK = (V + BC - 1) // BC
VPAD = NBLK * BC


def _format_table(emb):
  embT = emb.T

  def body(in_ref, out_ref):
    t = in_ref[...]
    u = lax.bitcast_convert_type(t, jnp.uint32)
    b = (u + jnp.uint32(0x7FFF) + ((u >> 16) & jnp.uint32(1))) >> 16
    neg = jnp.uint32(0) - (b >> 15)
    m = b ^ (jnp.uint32(0x8000) | (neg & jnp.uint32(0x7FFF)))
    packed = m[:DP] | (m[DP:] << 16)
    t8 = jnp.concatenate(
        [packed[:, CH * k:CH * (k + 1)] for k in range(8)], axis=0)
    out_ref[...] = t8.T[None]

  out = pl.pallas_call(
      body,
      grid=(NBLK,),
      in_specs=[pl.BlockSpec((D, BC), lambda i: (0, i))],
      out_specs=pl.BlockSpec((1, CH, 128), lambda i: (i, 0, 0)),
      out_shape=jax.ShapeDtypeStruct((NBLK, CH, 128), jnp.uint32),
  )(embT)
  return out.reshape(VPAD, DP)


def _remap(i):
  return (i & -8192) | ((i & 1023) << 3) | ((i >> 10) & 7)


def _maxpool_sc(xf, table):
  mesh = plsc.VectorSubcoreMesh(core_axis_name="c", subcore_axis_name="s")

  @functools.partial(
      pl.kernel,
      out_type=jax.ShapeDtypeStruct((B, DP), jnp.uint32),
      mesh=mesh,
      scratch_types=[
          pltpu.VMEM((2, GL), jnp.int32),
          pltpu.VMEM((2, G, L, DP), jnp.uint32),
          pltpu.VMEM((ROWS_PER_W, DP), jnp.uint32),
          pltpu.SemaphoreType.DMA,
          pltpu.SemaphoreType.DMA,
      ],
      compiler_params=pltpu.CompilerParams(use_tc_tiling_on_sc=False),
  )
  def k(xf_hbm, tab_hbm, out_hbm, idx_v, rows_v, out_v, sem0, sem1):
    wid = lax.axis_index("s") * NC + lax.axis_index("c")
    base = wid * ROWS_PER_W
    fbase = base * L
    sems = (sem0, sem1)

    def fire(g, p):
      sem = sems[p]
      pltpu.sync_copy(xf_hbm.at[pl.ds(fbase + g * GL, GL)], idx_v.at[p])

      def tbody(t, carry):
        for u in range(5):
          o = (t * 5 + u) * LANES
          idx_v[p, pl.ds(o, LANES)] = _remap(idx_v[p, pl.ds(o, LANES)])
        return carry

      lax.fori_loop(0, NVEC // 5, tbody, 0)
      for r in range(G):
        pltpu.async_copy(
            tab_hbm.at[idx_v.at[p, pl.ds(r * L, SPLIT0)]],
            rows_v.at[p, r, pl.ds(0, SPLIT0)], sem)
        pltpu.async_copy(
            tab_hbm.at[idx_v.at[p, pl.ds(r * L + SPLIT0, SPLIT1)]],
            rows_v.at[p, r, pl.ds(SPLIT0, SPLIT1)], sem)

    def drain(p):
      sem = sems[p]
      for r in range(G):
        pltpu.make_async_copy(
            tab_hbm.at[idx_v.at[p, pl.ds(r * L, SPLIT0)]],
            rows_v.at[p, r, pl.ds(0, SPLIT0)], sem).wait()
        pltpu.make_async_copy(
            tab_hbm.at[idx_v.at[p, pl.ds(r * L + SPLIT0, SPLIT1)]],
            rows_v.at[p, r, pl.ds(SPLIT0, SPLIT1)], sem).wait()

    zero = jnp.zeros((DP,), jnp.uint32)

    def compute(g, p):
      for r in range(G):
        def jbody(t, accs, r=r):
          ahi, alo = accs
          j = t * UNROLL
          for u in range(UNROLL):
            v = rows_v[p, r, j + u]
            ahi = jnp.maximum(ahi, v)
            alo = jnp.maximum(alo, v << 16)
          return (ahi, alo)

        ahi, alo = lax.fori_loop(0, L // UNROLL, jbody, (zero, zero))
        out_v[g * G + r] = (ahi & jnp.uint32(0xFFFF0000)) | (alo >> 16)

    fire(0, 0)

    def pair_body(gg, carry):
      g0 = 2 * gg
      fire(g0 + 1, 1)
      drain(0)
      compute(g0, 0)

      @pl.when(g0 + 2 < NG)
      def _():
        fire(g0 + 2, 0)

      drain(1)
      compute(g0 + 1, 1)
      return carry

    lax.fori_loop(0, NG // 2, pair_body, 0)
    pltpu.sync_copy(out_v, out_hbm.at[pl.ds(base, ROWS_PER_W)])

  return k(xf, table)


def _unkey(m):
  t = m >> 15
  mask = jnp.uint32(0xFFFF) ^ ((jnp.uint32(0) - t) & jnp.uint32(0x7FFF))
  return lax.bitcast_convert_type((m ^ mask) << 16, jnp.float32)


def _head_tc(set8, W_h, b_h, W_cls, b_cls, W_reg, b_reg):
  B8 = B // 8

  def body(s_ref, wh_ref, bh_ref, wc_ref, bc_ref, wr_ref, br_ref,
           cls_ref, reg_ref):
    sp = s_ref[...]
    cs, rs = [], []
    for k in range(8):
      zp = sp[:, DP * k:DP * (k + 1)]
      lo = _unkey(zp & jnp.uint32(0xFFFF))
      hi = _unkey(zp >> 16)
      z = jnp.concatenate([lo, hi], axis=1)
      h = jnp.dot(z, wh_ref[...], preferred_element_type=jnp.float32)
      h = h + bh_ref[...]
      h = jnp.where(h > 0, h, 0.01 * h)
      c = jnp.dot(h, wc_ref[...], preferred_element_type=jnp.float32)
      cs.append(jax.nn.sigmoid(c + bc_ref[...]))
      r = jnp.dot(h, wr_ref[...], preferred_element_type=jnp.float32)
      rs.append(r + br_ref[...])
    cls_ref[...] = jnp.concatenate(cs, axis=1)
    reg_ref[...] = jnp.concatenate(rs, axis=1)

  cls, reg = pl.pallas_call(
      body,
      in_specs=[
          pl.BlockSpec((B8, 128), lambda: (0, 0)),
          pl.BlockSpec((D, D), lambda: (0, 0)),
          pl.BlockSpec((1, D), lambda: (0, 0)),
          pl.BlockSpec((D, 1), lambda: (0, 0)),
          pl.BlockSpec((1, 1), lambda: (0, 0)),
          pl.BlockSpec((D, 1), lambda: (0, 0)),
          pl.BlockSpec((1, 1), lambda: (0, 0)),
      ],
      out_specs=[
          pl.BlockSpec((B8, 8), lambda: (0, 0)),
          pl.BlockSpec((B8, 8), lambda: (0, 0)),
      ],
      out_shape=[
          jax.ShapeDtypeStruct((B8, 8), jnp.float32),
          jax.ShapeDtypeStruct((B8, 8), jnp.float32),
      ],
  )(set8, W_h, b_h.reshape(1, D), W_cls, b_cls.reshape(1, 1),
    W_reg, b_reg.reshape(1, 1))
  return cls.reshape(B, 1), reg.reshape(B, 1)


def kernel(x, emb, W_h, b_h, W_cls, b_cls, W_reg, b_reg):
  table = _format_table(emb)
  xf = x.astype(jnp.int32).reshape(B * L)
  set_vec = _maxpool_sc(xf, table)
  set8 = set_vec.reshape(B // 8, 128)
  return _head_tc(set8, W_h, b_h, W_cls, b_cls, W_reg, b_reg)
